# baseline (speedup 1.0000x reference)
import functools

import jax
import jax.numpy as jnp
from jax import lax
from jax.experimental import pallas as pl
from jax.experimental.pallas import tpu as pltpu
from jax.experimental.pallas import tpu_sc as plsc

BATCH = 16384
EMBED = 64
HIDDEN = 128
N_LAB = 1000000
N_CAT = 100000
NC = 2
NS = 16
NW = NC * NS
B_PER_W = BATCH // NW
CHUNK = 128
N_CHUNK = B_PER_W // CHUNK
BLK = 4096
PBLK = 8192

LAB_OFF = 61
LAB_SPLIT = LAB_OFF * PBLK
LAB_OUT = N_LAB - LAB_SPLIT
LAB_MOFF = 30
LAB_M = LAB_MOFF * PBLK
LAB_W = LAB_OUT - LAB_M
CAT_OFF = 6
CAT_SPLIT = CAT_OFF * PBLK
CAT_OUT = N_CAT - CAT_SPLIT
CAT_MOFF = 3
CAT_M = CAT_MOFF * PBLK
CAT_W = CAT_OUT - CAT_M


def _pack_body(a1_ref, b1_ref, a2_ref, b2_ref, out_ref):
    eye = jnp.eye(EMBED, dtype=jnp.float32)

    def tr(r):
        return lax.dot_general(r[...], eye, (((0,), (0,)), ((), ())),
                               preferred_element_type=jnp.float32)

    low = jnp.concatenate([tr(a1_ref), tr(b1_ref)], axis=1)
    high = jnp.concatenate([tr(a2_ref), tr(b2_ref)], axis=1)
    lo16 = lax.bitcast_convert_type(low.astype(jnp.bfloat16), jnp.int16)
    hi16 = lax.bitcast_convert_type(high.astype(jnp.bfloat16), jnp.int16)
    out_ref[...] = ((hi16.astype(jnp.int32) << 16)
                    | (lo16.astype(jnp.int32) & 0xFFFF))


def _pack(tT, n_words, boff, moff):
    grid = (n_words + PBLK - 1) // PBLK
    return pl.pallas_call(
        _pack_body,
        grid=(grid,),
        in_specs=[
            pl.BlockSpec((EMBED, PBLK), lambda i: (0, i)),
            pl.BlockSpec((EMBED, PBLK), lambda i, _b=boff: (0, i + _b)),
            pl.BlockSpec((EMBED, PBLK), lambda i, _m=moff: (0, i + _m)),
            pl.BlockSpec((EMBED, PBLK),
                         lambda i, _bm=boff + moff: (0, i + _bm)),
        ],
        out_specs=pl.BlockSpec((PBLK, 2 * EMBED), lambda i: (i, 0)),
        out_shape=jax.ShapeDtypeStruct((n_words, 2 * EMBED), jnp.int32),
    )(tT, tT, tT, tT)


def _gather_body(lidx_hbm, cidx_hbm, ltab_hbm, ctab_hbm, el_out, ec_out,
                 idx_l, idx_c, rows, sem):
    wid = lax.axis_index("s") * NC + lax.axis_index("c")
    base_row = wid * N_CHUNK
    pltpu.sync_copy(lidx_hbm.at[pl.ds(base_row, N_CHUNK)], idx_l)
    pltpu.sync_copy(cidx_hbm.at[pl.ds(base_row, N_CHUNK)], idx_c)
    base = wid * B_PER_W
    copies = []
    for j in range(N_CHUNK):
        copies.append(pltpu.async_copy(
            ltab_hbm.at[idx_l.at[j]], rows.at[pl.ds(j * CHUNK, CHUNK)], sem))
    for c in copies:
        c.wait()
    pltpu.sync_copy(rows, el_out.at[pl.ds(base, B_PER_W)])
    copies = []
    for j in range(N_CHUNK):
        copies.append(pltpu.async_copy(
            ctab_hbm.at[idx_c.at[j]], rows.at[pl.ds(j * CHUNK, CHUNK)], sem))
    for c in copies:
        c.wait()
    pltpu.sync_copy(rows, ec_out.at[pl.ds(base, B_PER_W)])


@functools.lru_cache(maxsize=None)
def _make_gather():
    return pl.kernel(
        _gather_body,
        mesh=plsc.VectorSubcoreMesh(core_axis_name="c", subcore_axis_name="s"),
        out_type=[
            jax.ShapeDtypeStruct((BATCH, 2 * EMBED), jnp.int32),
            jax.ShapeDtypeStruct((BATCH, 2 * EMBED), jnp.int32),
        ],
        scratch_types=[
            pltpu.VMEM((N_CHUNK, CHUNK), jnp.int32),
            pltpu.VMEM((N_CHUNK, CHUNK), jnp.int32),
            pltpu.VMEM((B_PER_W, 2 * EMBED), jnp.int32),
            pltpu.SemaphoreType.DMA,
        ],
        compiler_params=pltpu.CompilerParams(use_tc_tiling_on_sc=True),
    )


def _mlp_body(el_ref, ec_ref, pl_ref, pc_ref, hl_ref, hc_ref,
              w1a_ref, w1b_ref, b1_ref,
              w2_ref, b2_ref, w3_ref, b3_ref, out_ref):
    wl = el_ref[...]
    wc = ec_ref[...]
    vl = jnp.where(hl_ref[...] > 0,
                   wl & jnp.int32(-65536), wl << 16)
    vc = jnp.where(hc_ref[...] > 0,
                   wc & jnp.int32(-65536), wc << 16)
    el2 = lax.bitcast_convert_type(vl, jnp.float32)
    ec2 = lax.bitcast_convert_type(vc, jnp.float32)
    el = jnp.where(pl_ref[...] > 0, el2[:, EMBED:], el2[:, :EMBED])
    ec = jnp.where(pc_ref[...] > 0, ec2[:, EMBED:], ec2[:, :EMBED])
    h = jnp.dot(el, w1a_ref[...], preferred_element_type=jnp.float32)
    h = h + jnp.dot(ec, w1b_ref[...], preferred_element_type=jnp.float32)
    h = jnp.maximum(h + b1_ref[...], 0.0)
    h = jnp.maximum(
        jnp.dot(h, w2_ref[...], preferred_element_type=jnp.float32) + b2_ref[...],
        0.0)
    out_ref[...] = (
        jnp.dot(h, w3_ref[...], preferred_element_type=jnp.float32) + b3_ref[...])


def _mlp(el, ec, par_l, par_c, h_l, h_c, W1a, W1b, b1, W2, b2, W3, b3):
    return pl.pallas_call(
        _mlp_body,
        grid=(BATCH // BLK,),
        in_specs=[
            pl.BlockSpec((BLK, 2 * EMBED), lambda i: (i, 0)),
            pl.BlockSpec((BLK, 2 * EMBED), lambda i: (i, 0)),
            pl.BlockSpec((BLK, 1), lambda i: (i, 0)),
            pl.BlockSpec((BLK, 1), lambda i: (i, 0)),
            pl.BlockSpec((BLK, 1), lambda i: (i, 0)),
            pl.BlockSpec((BLK, 1), lambda i: (i, 0)),
            pl.BlockSpec((EMBED, HIDDEN), lambda i: (0, 0)),
            pl.BlockSpec((EMBED, HIDDEN), lambda i: (0, 0)),
            pl.BlockSpec((1, HIDDEN), lambda i: (0, 0)),
            pl.BlockSpec((HIDDEN, HIDDEN), lambda i: (0, 0)),
            pl.BlockSpec((1, HIDDEN), lambda i: (0, 0)),
            pl.BlockSpec((HIDDEN, 2), lambda i: (0, 0)),
            pl.BlockSpec((1, 2), lambda i: (0, 0)),
        ],
        out_specs=pl.BlockSpec((BLK, 2), lambda i: (i, 0)),
        out_shape=jax.ShapeDtypeStruct((BATCH, 2), jnp.float32),
    )(el, ec, par_l, par_c, h_l, h_c, W1a, W1b, b1, W2, b2, W3, b3)


def kernel(label_idx, category_idx, label_table, category_table,
           W1, b1, W2, b2, W3, b3):
    lidx = label_idx.astype(jnp.int32)
    cidx = category_idx.astype(jnp.int32)
    lsel = (lidx >= LAB_SPLIT).astype(jnp.int32)
    csel = (cidx >= CAT_SPLIT).astype(jnp.int32)
    lchunk = lidx - lsel * LAB_SPLIT
    cchunk = cidx - csel * CAT_SPLIT
    lhi = (lchunk >= LAB_M).astype(jnp.int32)
    chi = (cchunk >= CAT_M).astype(jnp.int32)
    lword = (lchunk - lhi * LAB_M).reshape(BATCH // CHUNK, CHUNK)
    cword = (cchunk - chi * CAT_M).reshape(BATCH // CHUNK, CHUNK)
    ltp = _pack(label_table.T, LAB_W, LAB_OFF, LAB_MOFF)
    ctp = _pack(category_table.T, CAT_W, CAT_OFF, CAT_MOFF)
    el, ec = _make_gather()(lword, cword, ltp, ctp)
    return _mlp(el, ec, lsel.reshape(BATCH, 1), csel.reshape(BATCH, 1),
                lhi.reshape(BATCH, 1), chi.reshape(BATCH, 1),
                W1[:EMBED], W1[EMBED:],
                b1.reshape(1, HIDDEN), W2, b2.reshape(1, HIDDEN),
                W3, b3.reshape(1, 2))


# speedup vs baseline: 1.0184x; 1.0184x over previous
import functools

import jax
import jax.numpy as jnp
from jax import lax
from jax.experimental import pallas as pl
from jax.experimental.pallas import tpu as pltpu
from jax.experimental.pallas import tpu_sc as plsc

BATCH = 16384
EMBED = 64
HIDDEN = 128
N_LAB = 1000000
N_CAT = 100000
NC = 2
NS = 16
NW = NC * NS
B_PER_W = BATCH // NW
CHUNK = 128
N_CHUNK = B_PER_W // CHUNK
BLK = 2048
PBLK = 4096

LAB_OFF = 122
LAB_SPLIT = LAB_OFF * PBLK
LAB_OUT = N_LAB - LAB_SPLIT
LAB_MOFF = 61
LAB_M = LAB_MOFF * PBLK
LAB_W = LAB_OUT - LAB_M
CAT_OFF = 12
CAT_SPLIT = CAT_OFF * PBLK
CAT_OUT = N_CAT - CAT_SPLIT
CAT_MOFF = 6
CAT_M = CAT_MOFF * PBLK
CAT_W = CAT_OUT - CAT_M


def _pack_body(a1_ref, b1_ref, a2_ref, b2_ref, out_ref):
    eye = jnp.eye(EMBED, dtype=jnp.float32)

    def tr(r):
        return lax.dot_general(r[...], eye, (((0,), (0,)), ((), ())),
                               preferred_element_type=jnp.float32)

    low = jnp.concatenate([tr(a1_ref), tr(b1_ref)], axis=1)
    high = jnp.concatenate([tr(a2_ref), tr(b2_ref)], axis=1)
    lo16 = lax.bitcast_convert_type(low.astype(jnp.bfloat16), jnp.int16)
    hi16 = lax.bitcast_convert_type(high.astype(jnp.bfloat16), jnp.int16)
    out_ref[...] = ((hi16.astype(jnp.int32) << 16)
                    | (lo16.astype(jnp.int32) & 0xFFFF))


def _pack(tT, n_words, boff, moff):
    grid = (n_words + PBLK - 1) // PBLK
    return pl.pallas_call(
        _pack_body,
        grid=(grid,),
        in_specs=[
            pl.BlockSpec((EMBED, PBLK), lambda i: (0, i)),
            pl.BlockSpec((EMBED, PBLK), lambda i, _b=boff: (0, i + _b)),
            pl.BlockSpec((EMBED, PBLK), lambda i, _m=moff: (0, i + _m)),
            pl.BlockSpec((EMBED, PBLK),
                         lambda i, _bm=boff + moff: (0, i + _bm)),
        ],
        out_specs=pl.BlockSpec((PBLK, 2 * EMBED), lambda i: (i, 0)),
        out_shape=jax.ShapeDtypeStruct((n_words, 2 * EMBED), jnp.int32),
    )(tT, tT, tT, tT)


def _gather_body(lidx_hbm, cidx_hbm, ltab_hbm, ctab_hbm, el_out, ec_out,
                 idx_l, idx_c, rows, sem):
    wid = lax.axis_index("s") * NC + lax.axis_index("c")
    base_row = wid * N_CHUNK
    pltpu.sync_copy(lidx_hbm.at[pl.ds(base_row, N_CHUNK)], idx_l)
    pltpu.sync_copy(cidx_hbm.at[pl.ds(base_row, N_CHUNK)], idx_c)
    base = wid * B_PER_W
    copies = []
    for j in range(N_CHUNK):
        copies.append(pltpu.async_copy(
            ltab_hbm.at[idx_l.at[j]], rows.at[pl.ds(j * CHUNK, CHUNK)], sem))
    for c in copies:
        c.wait()
    pltpu.sync_copy(rows, el_out.at[pl.ds(base, B_PER_W)])
    copies = []
    for j in range(N_CHUNK):
        copies.append(pltpu.async_copy(
            ctab_hbm.at[idx_c.at[j]], rows.at[pl.ds(j * CHUNK, CHUNK)], sem))
    for c in copies:
        c.wait()
    pltpu.sync_copy(rows, ec_out.at[pl.ds(base, B_PER_W)])


@functools.lru_cache(maxsize=None)
def _make_gather():
    return pl.kernel(
        _gather_body,
        mesh=plsc.VectorSubcoreMesh(core_axis_name="c", subcore_axis_name="s"),
        out_type=[
            jax.ShapeDtypeStruct((BATCH, 2 * EMBED), jnp.int32),
            jax.ShapeDtypeStruct((BATCH, 2 * EMBED), jnp.int32),
        ],
        scratch_types=[
            pltpu.VMEM((N_CHUNK, CHUNK), jnp.int32),
            pltpu.VMEM((N_CHUNK, CHUNK), jnp.int32),
            pltpu.VMEM((B_PER_W, 2 * EMBED), jnp.int32),
            pltpu.SemaphoreType.DMA,
        ],
        compiler_params=pltpu.CompilerParams(use_tc_tiling_on_sc=True),
    )


def _mlp_body(el_ref, ec_ref, pl_ref, pc_ref, hl_ref, hc_ref,
              w1a_ref, w1b_ref, b1_ref,
              w2_ref, b2_ref, w3_ref, b3_ref, out_ref):
    wl = el_ref[...]
    wc = ec_ref[...]
    vl = jnp.where(hl_ref[...] > 0,
                   wl & jnp.int32(-65536), wl << 16)
    vc = jnp.where(hc_ref[...] > 0,
                   wc & jnp.int32(-65536), wc << 16)
    el2 = lax.bitcast_convert_type(vl, jnp.float32)
    ec2 = lax.bitcast_convert_type(vc, jnp.float32)
    el = jnp.where(pl_ref[...] > 0, el2[:, EMBED:], el2[:, :EMBED])
    ec = jnp.where(pc_ref[...] > 0, ec2[:, EMBED:], ec2[:, :EMBED])
    el = el.astype(jnp.bfloat16)
    ec = ec.astype(jnp.bfloat16)
    h = jnp.dot(el, w1a_ref[...], preferred_element_type=jnp.float32)
    h = h + jnp.dot(ec, w1b_ref[...], preferred_element_type=jnp.float32)
    h = jnp.maximum(h + b1_ref[...], 0.0)
    h = jnp.maximum(
        jnp.dot(h, w2_ref[...], preferred_element_type=jnp.float32) + b2_ref[...],
        0.0)
    out_ref[...] = (
        jnp.dot(h, w3_ref[...], preferred_element_type=jnp.float32) + b3_ref[...])


def _mlp(el, ec, par_l, par_c, h_l, h_c, W1a, W1b, b1, W2, b2, W3, b3):
    return pl.pallas_call(
        _mlp_body,
        grid=(BATCH // BLK,),
        in_specs=[
            pl.BlockSpec((BLK, 2 * EMBED), lambda i: (i, 0)),
            pl.BlockSpec((BLK, 2 * EMBED), lambda i: (i, 0)),
            pl.BlockSpec((BLK, 1), lambda i: (i, 0)),
            pl.BlockSpec((BLK, 1), lambda i: (i, 0)),
            pl.BlockSpec((BLK, 1), lambda i: (i, 0)),
            pl.BlockSpec((BLK, 1), lambda i: (i, 0)),
            pl.BlockSpec((EMBED, HIDDEN), lambda i: (0, 0)),
            pl.BlockSpec((EMBED, HIDDEN), lambda i: (0, 0)),
            pl.BlockSpec((1, HIDDEN), lambda i: (0, 0)),
            pl.BlockSpec((HIDDEN, HIDDEN), lambda i: (0, 0)),
            pl.BlockSpec((1, HIDDEN), lambda i: (0, 0)),
            pl.BlockSpec((HIDDEN, 2), lambda i: (0, 0)),
            pl.BlockSpec((1, 2), lambda i: (0, 0)),
        ],
        out_specs=pl.BlockSpec((BLK, 2), lambda i: (i, 0)),
        out_shape=jax.ShapeDtypeStruct((BATCH, 2), jnp.float32),
    )(el, ec, par_l, par_c, h_l, h_c, W1a, W1b, b1, W2, b2, W3, b3)


def kernel(label_idx, category_idx, label_table, category_table,
           W1, b1, W2, b2, W3, b3):
    lidx = label_idx.astype(jnp.int32)
    cidx = category_idx.astype(jnp.int32)
    lsel = (lidx >= LAB_SPLIT).astype(jnp.int32)
    csel = (cidx >= CAT_SPLIT).astype(jnp.int32)
    lchunk = lidx - lsel * LAB_SPLIT
    cchunk = cidx - csel * CAT_SPLIT
    lhi = (lchunk >= LAB_M).astype(jnp.int32)
    chi = (cchunk >= CAT_M).astype(jnp.int32)
    lword = (lchunk - lhi * LAB_M).reshape(BATCH // CHUNK, CHUNK)
    cword = (cchunk - chi * CAT_M).reshape(BATCH // CHUNK, CHUNK)
    ltp = _pack(label_table.T, LAB_W, LAB_OFF, LAB_MOFF)
    ctp = _pack(category_table.T, CAT_W, CAT_OFF, CAT_MOFF)
    el, ec = _make_gather()(lword, cword, ltp, ctp)
    return _mlp(el, ec, lsel.reshape(BATCH, 1), csel.reshape(BATCH, 1),
                lhi.reshape(BATCH, 1), chi.reshape(BATCH, 1),
                W1[:EMBED].astype(jnp.bfloat16), W1[EMBED:].astype(jnp.bfloat16),
                b1.reshape(1, HIDDEN), W2, b2.reshape(1, HIDDEN),
                W3, b3.reshape(1, 2))
